# Initial kernel scaffold; baseline (speedup 1.0000x reference)
import jax
import jax.numpy as jnp
from jax.experimental import pallas as pl


def kernel(x, w_b1_0, b_b1_0, w_b1_1, b_b1_1, w_b1_2, b_b1_2, w_b2_0, b_b2_0, w_b2_1, b_b2_1, w_b2_2, b_b2_2):
    raise NotImplementedError("write your pallas kernel here")


# speedup vs baseline: 1.7997x; 1.7997x over previous
import functools

import numpy as np
import jax
import jax.numpy as jnp
from jax.experimental import pallas as pl
from jax.experimental.pallas import tpu as pltpu


def _round_up(n, m):
    return ((n + m - 1) // m) * m


def _compact_kernel(x_ref, w_ref, b_ref, mi_ref, mo_ref, o_ref, *,
                    E, N, S, W, CH, C2, KD, KH, KW, layers_per_block,
                    n_blocks):
    NT = E * N
    pd, ph, pw = KD // 2, KH // 2, KW // 2

    def shift(v, s):
        if s % NT == 0:
            return v
        return pltpu.roll(v, (-s) % NT, axis=1)

    mi = [mi_ref[k:k + 1, :] for k in range(KD * KW)]
    mo = [mo_ref[k:k + 1, :] for k in range(KH)]

    h = jnp.concatenate(
        [x_ref[...], jnp.zeros((CH - C2, NT), jnp.float32)], axis=0)

    li = 0
    for _blk in range(n_blocks):
        h_in = h
        for _l in range(layers_per_block):
            hb = h.astype(jnp.bfloat16)
            zs = []
            for kd in range(KD):
                for kw in range(KW):
                    v = shift(hb, (kd - pd) * S + (kw - pw))
                    if kd != pd or kw != pw:
                        v = v * mi[kd * KW + kw]
                    zs.append(v)
            z = jnp.concatenate(zs, axis=0)
            r = jnp.dot(w_ref[li], z,
                        preferred_element_type=jnp.float32)
            acc = None
            for kh in range(KH):
                part = shift(r[kh * CH:(kh + 1) * CH, :], (kh - ph) * W)
                if kh != ph:
                    part = part * mo[kh]
                acc = part if acc is None else acc + part
            h = jnp.maximum(acc + b_ref[li], 0.0)
            li += 1
        h = h + h_in
    o_ref[...] = h[:C2, :]


def _forward(x, layers):
    B, D, H, W, C2 = x.shape
    KD, KH, KW = layers[0][0].shape[:3]
    pd, ph, pw = KD // 2, KH // 2, KW // 2
    CH = _round_up(max(max(w.shape[3], w.shape[4]) for w, _ in layers), 8)
    S, N = H * W, D * H * W
    E = 2 if B % 2 == 0 else 1
    NT = E * N

    xt = jnp.moveaxis(x, -1, 1).reshape(B // E, E, C2, N)
    x_flat = jnp.transpose(xt, (0, 2, 1, 3)).reshape(B // E, C2, NT)

    w_stack, b_stack = [], []
    for (w, b) in layers:
        cin, cout = w.shape[3], w.shape[4]
        wpd = jnp.zeros((KD, KH, KW, CH, CH), jnp.float32)
        wpd = wpd.at[..., :cin, :cout].set(w)
        w_stack.append(
            jnp.transpose(wpd, (1, 4, 0, 2, 3)).reshape(KH * CH, KD * KW * CH))
        b_stack.append(
            jnp.zeros((CH,), jnp.float32).at[:cout].set(b).reshape(CH, 1))
    w_stack = jnp.stack(w_stack).astype(jnp.bfloat16)
    b_stack = jnp.stack(b_stack)

    q = np.arange(NT)
    qd = (q % N) // S
    qh = (q % S) // W
    qw = q % W
    mi = np.zeros((16, NT), np.float32)
    for kd in range(KD):
        for kw in range(KW):
            mi[kd * KW + kw] = (
                (qd + kd - pd >= 0) & (qd + kd - pd < D)
                & (qw + kw - pw >= 0) & (qw + kw - pw < W))
    mo = np.zeros((8, NT), np.float32)
    for kh in range(KH):
        mo[kh] = (qh + kh - ph >= 0) & (qh + kh - ph < H)
    mi = jnp.asarray(mi, jnp.bfloat16)
    mo = jnp.asarray(mo)

    kfn = functools.partial(
        _compact_kernel, E=E, N=N, S=S, W=W, CH=CH, C2=C2, KD=KD, KH=KH,
        KW=KW, layers_per_block=len(layers) // 2, n_blocks=2)

    out = pl.pallas_call(
        kfn,
        out_shape=jax.ShapeDtypeStruct((B // E, C2, NT), jnp.float32),
        grid_spec=pltpu.PrefetchScalarGridSpec(
            num_scalar_prefetch=0,
            grid=(B // E,),
            in_specs=[
                pl.BlockSpec((None, C2, NT), lambda g: (g, 0, 0)),
                pl.BlockSpec(w_stack.shape, lambda g: (0, 0, 0)),
                pl.BlockSpec(b_stack.shape, lambda g: (0, 0, 0)),
                pl.BlockSpec(mi.shape, lambda g: (0, 0)),
                pl.BlockSpec(mo.shape, lambda g: (0, 0)),
            ],
            out_specs=pl.BlockSpec((None, C2, NT), lambda g: (g, 0, 0)),
        ),
        compiler_params=pltpu.CompilerParams(
            dimension_semantics=("parallel",),
            vmem_limit_bytes=64 * 1024 * 1024),
    )(x_flat, w_stack, b_stack, mi, mo)

    out = out.reshape(B // E, C2, E, N)
    out = jnp.transpose(out, (0, 2, 1, 3)).reshape(B, C2, D, H, W)
    return jnp.moveaxis(out, 1, -1)


def kernel(x, w_b1_0, b_b1_0, w_b1_1, b_b1_1, w_b1_2, b_b1_2,
           w_b2_0, b_b2_0, w_b2_1, b_b2_1, w_b2_2, b_b2_2):
    layers = [(w_b1_0, b_b1_0), (w_b1_1, b_b1_1), (w_b1_2, b_b1_2),
              (w_b2_0, b_b2_0), (w_b2_1, b_b2_1), (w_b2_2, b_b2_2)]
    return _forward(x, layers)


# speedup vs baseline: 3.7629x; 2.0909x over previous
import functools

import numpy as np
import jax
import jax.numpy as jnp
from jax.experimental import pallas as pl
from jax.experimental.pallas import tpu as pltpu


def _round_up(n, m):
    return ((n + m - 1) // m) * m


def _compact_kernel(x_ref, w_ref, b_ref, mw_ref, mo_ref, o_ref, z_ref, *,
                    E, N, S, W, D, CH, C2, KD, KH, KW, layers_per_block,
                    n_blocks):
    SL = E * S
    NT = D * SL
    pd, ph, pw = KD // 2, KH // 2, KW // 2

    def roll(v, s, width):
        if s % width == 0:
            return v
        return pltpu.roll(v, (-s) % width, axis=1)

    mw = [mw_ref[k:k + 1, :] for k in range(KW)]
    mo = [mo_ref[k:k + 1, :] for k in range(KH)]

    for kd in range(KD):
        if kd < pd:
            z_ref[kd * KW * CH:(kd + 1) * KW * CH, :(pd - kd) * SL] = (
                jnp.zeros((KW * CH, (pd - kd) * SL), jnp.bfloat16))
        elif kd > pd:
            z_ref[kd * KW * CH:(kd + 1) * KW * CH, NT - (kd - pd) * SL:] = (
                jnp.zeros((KW * CH, (kd - pd) * SL), jnp.bfloat16))

    h = jnp.concatenate(
        [x_ref[...], jnp.zeros((CH - C2, NT), jnp.float32)], axis=0)

    li = 0
    for _blk in range(n_blocks):
        h_in = h
        for _l in range(layers_per_block):
            hb = h.astype(jnp.bfloat16)
            for src in range(D):
                sl = hb[:, src * SL:(src + 1) * SL]
                for kw in range(KW):
                    if kw == pw:
                        u = sl
                    else:
                        u = roll(sl, kw - pw, SL) * mw[kw]
                    row = kw * CH
                    for kd in range(KD):
                        d = src - kd + pd
                        if 0 <= d < D:
                            z_ref[kd * KW * CH + row:
                                  kd * KW * CH + row + CH,
                                  d * SL:(d + 1) * SL] = u
            r = jnp.dot(w_ref[li], z_ref[...],
                        preferred_element_type=jnp.float32)
            acc = None
            for kh in range(KH):
                part = roll(r[kh * CH:(kh + 1) * CH, :], (kh - ph) * W, NT)
                if kh != ph:
                    part = part * mo[kh]
                acc = part if acc is None else acc + part
            h = jnp.maximum(acc + b_ref[li], 0.0)
            li += 1
        h = h + h_in
    o_ref[...] = h[:C2, :]


def _forward(x, layers):
    B, D, H, W, C2 = x.shape
    KD, KH, KW = layers[0][0].shape[:3]
    pd, ph, pw = KD // 2, KH // 2, KW // 2
    CH = _round_up(max(max(w.shape[3], w.shape[4]) for w, _ in layers), 8)
    S, N = H * W, D * H * W
    E = next((e for e in (8, 16, 4, 2, 1)
              if B % e == 0 and (e * S) % 128 == 0), 1)
    SL = E * S
    NT = D * SL

    xt = jnp.moveaxis(x, -1, 1).reshape(B // E, E, C2, D, S)
    x_flat = jnp.transpose(xt, (0, 2, 3, 1, 4)).reshape(B // E, C2, NT)

    w_stack, b_stack = [], []
    for (w, b) in layers:
        cin, cout = w.shape[3], w.shape[4]
        wpd = jnp.zeros((KD, KH, KW, CH, CH), jnp.float32)
        wpd = wpd.at[..., :cin, :cout].set(w)
        w_stack.append(
            jnp.transpose(wpd, (1, 4, 0, 2, 3)).reshape(KH * CH, KD * KW * CH))
        b_stack.append(
            jnp.zeros((CH,), jnp.float32).at[:cout].set(b).reshape(CH, 1))
    w_stack = jnp.stack(w_stack).astype(jnp.bfloat16)
    b_stack = jnp.stack(b_stack)

    qs = np.arange(SL)
    qw_s = qs % W
    mw = np.zeros((8, SL), np.float32)
    for kw in range(KW):
        mw[kw] = (qw_s + kw - pw >= 0) & (qw_s + kw - pw < W)
    q = np.arange(NT)
    qh = (q % S) // W
    mo = np.zeros((8, NT), np.float32)
    for kh in range(KH):
        mo[kh] = (qh + kh - ph >= 0) & (qh + kh - ph < H)
    mw = jnp.asarray(mw, jnp.bfloat16)
    mo = jnp.asarray(mo)

    kfn = functools.partial(
        _compact_kernel, E=E, N=N, S=S, W=W, D=D, CH=CH, C2=C2, KD=KD,
        KH=KH, KW=KW, layers_per_block=len(layers) // 2, n_blocks=2)

    out = pl.pallas_call(
        kfn,
        out_shape=jax.ShapeDtypeStruct((B // E, C2, NT), jnp.float32),
        grid_spec=pltpu.PrefetchScalarGridSpec(
            num_scalar_prefetch=0,
            grid=(B // E,),
            in_specs=[
                pl.BlockSpec((None, C2, NT), lambda g: (g, 0, 0)),
                pl.BlockSpec(w_stack.shape, lambda g: (0, 0, 0)),
                pl.BlockSpec(b_stack.shape, lambda g: (0, 0, 0)),
                pl.BlockSpec(mw.shape, lambda g: (0, 0)),
                pl.BlockSpec(mo.shape, lambda g: (0, 0)),
            ],
            out_specs=pl.BlockSpec((None, C2, NT), lambda g: (g, 0, 0)),
            scratch_shapes=[
                pltpu.VMEM((KD * KW * CH, NT), jnp.bfloat16)],
        ),
        compiler_params=pltpu.CompilerParams(
            dimension_semantics=("parallel",),
            vmem_limit_bytes=56 * 1024 * 1024),
    )(x_flat, w_stack, b_stack, mw, mo)

    out = out.reshape(B // E, C2, D, E, S)
    out = jnp.transpose(out, (0, 3, 1, 2, 4)).reshape(B, C2, D, H, W)
    return jnp.moveaxis(out, 1, -1)


def kernel(x, w_b1_0, b_b1_0, w_b1_1, b_b1_1, w_b1_2, b_b1_2,
           w_b2_0, b_b2_0, w_b2_1, b_b2_1, w_b2_2, b_b2_2):
    layers = [(w_b1_0, b_b1_0), (w_b1_1, b_b1_1), (w_b1_2, b_b1_2),
              (w_b2_0, b_b2_0), (w_b2_1, b_b2_1), (w_b2_2, b_b2_2)]
    return _forward(x, layers)


# speedup vs baseline: 3.7679x; 1.0013x over previous
import functools

import numpy as np
import jax
import jax.numpy as jnp
from jax.experimental import pallas as pl
from jax.experimental.pallas import tpu as pltpu


def _round_up(n, m):
    return ((n + m - 1) // m) * m


def _compact_kernel(x_ref, w_ref, b_ref, mw_ref, mo_ref, o_ref, z_ref, *,
                    E, N, S, W, D, CH, C2, KD, KH, KW, layers_per_block,
                    n_blocks):
    SL = E * S
    NT = D * SL
    pd, ph, pw = KD // 2, KH // 2, KW // 2

    def roll(v, s, width):
        if s % width == 0:
            return v
        return pltpu.roll(v, (-s) % width, axis=1)

    mw = [mw_ref[k:k + 1, :] for k in range(KW)]
    mo = [mo_ref[k:k + 1, :] for k in range(KH)]

    for kd in range(KD):
        if kd < pd:
            z_ref[kd * KW * CH:(kd + 1) * KW * CH, :(pd - kd) * SL] = (
                jnp.zeros((KW * CH, (pd - kd) * SL), jnp.bfloat16))
        elif kd > pd:
            z_ref[kd * KW * CH:(kd + 1) * KW * CH, NT - (kd - pd) * SL:] = (
                jnp.zeros((KW * CH, (kd - pd) * SL), jnp.bfloat16))

    h = jnp.concatenate(
        [x_ref[...], jnp.zeros((CH - C2, NT), jnp.float32)], axis=0)

    li = 0
    for _blk in range(n_blocks):
        h_in = h
        for _l in range(layers_per_block):
            hb = h.astype(jnp.bfloat16)
            for src in range(D):
                sl = hb[:, src * SL:(src + 1) * SL]
                for kw in range(KW):
                    if kw == pw:
                        u = sl
                    else:
                        u = roll(sl, kw - pw, SL) * mw[kw]
                    row = kw * CH
                    for kd in range(KD):
                        d = src - kd + pd
                        if 0 <= d < D:
                            z_ref[kd * KW * CH + row:
                                  kd * KW * CH + row + CH,
                                  d * SL:(d + 1) * SL] = u
            r = jnp.dot(w_ref[li], z_ref[...],
                        preferred_element_type=jnp.float32)
            acc = None
            for kh in range(KH):
                part = roll(r[kh * CH:(kh + 1) * CH, :], (kh - ph) * W, NT)
                if kh != ph:
                    part = part * mo[kh]
                acc = part if acc is None else acc + part
            h = jnp.maximum(acc + b_ref[li], 0.0)
            li += 1
        h = h + h_in
    o_ref[...] = h[:C2, :]


def _forward(x, layers):
    B, D, H, W, C2 = x.shape
    KD, KH, KW = layers[0][0].shape[:3]
    pd, ph, pw = KD // 2, KH // 2, KW // 2
    CH = _round_up(max(max(w.shape[3], w.shape[4]) for w, _ in layers), 8)
    S, N = H * W, D * H * W
    E = next((e for e in (8, 16, 4, 2, 1)
              if B % e == 0 and (e * S) % 128 == 0), 1)
    SL = E * S
    NT = D * SL

    x_flat = jnp.transpose(
        x.reshape(B // E, E, D, S, C2), (0, 4, 2, 1, 3)).reshape(
            B // E, C2, NT)

    w_stack, b_stack = [], []
    for (w, b) in layers:
        cin, cout = w.shape[3], w.shape[4]
        wpd = jnp.zeros((KD, KH, KW, CH, CH), jnp.float32)
        wpd = wpd.at[..., :cin, :cout].set(w)
        w_stack.append(
            jnp.transpose(wpd, (1, 4, 0, 2, 3)).reshape(KH * CH, KD * KW * CH))
        b_stack.append(
            jnp.zeros((CH,), jnp.float32).at[:cout].set(b).reshape(CH, 1))
    w_stack = jnp.stack(w_stack).astype(jnp.bfloat16)
    b_stack = jnp.stack(b_stack)

    qs = np.arange(SL)
    qw_s = qs % W
    mw = np.zeros((8, SL), np.float32)
    for kw in range(KW):
        mw[kw] = (qw_s + kw - pw >= 0) & (qw_s + kw - pw < W)
    q = np.arange(NT)
    qh = (q % S) // W
    mo = np.zeros((8, NT), np.float32)
    for kh in range(KH):
        mo[kh] = (qh + kh - ph >= 0) & (qh + kh - ph < H)
    mw = jnp.asarray(mw, jnp.bfloat16)
    mo = jnp.asarray(mo)

    kfn = functools.partial(
        _compact_kernel, E=E, N=N, S=S, W=W, D=D, CH=CH, C2=C2, KD=KD,
        KH=KH, KW=KW, layers_per_block=len(layers) // 2, n_blocks=2)

    out = pl.pallas_call(
        kfn,
        out_shape=jax.ShapeDtypeStruct((B // E, C2, NT), jnp.float32),
        grid_spec=pltpu.PrefetchScalarGridSpec(
            num_scalar_prefetch=0,
            grid=(B // E,),
            in_specs=[
                pl.BlockSpec((None, C2, NT), lambda g: (g, 0, 0)),
                pl.BlockSpec(w_stack.shape, lambda g: (0, 0, 0)),
                pl.BlockSpec(b_stack.shape, lambda g: (0, 0, 0)),
                pl.BlockSpec(mw.shape, lambda g: (0, 0)),
                pl.BlockSpec(mo.shape, lambda g: (0, 0)),
            ],
            out_specs=pl.BlockSpec((None, C2, NT), lambda g: (g, 0, 0)),
            scratch_shapes=[
                pltpu.VMEM((KD * KW * CH, NT), jnp.bfloat16)],
        ),
        compiler_params=pltpu.CompilerParams(
            dimension_semantics=("parallel",),
            vmem_limit_bytes=56 * 1024 * 1024),
    )(x_flat, w_stack, b_stack, mw, mo)

    out = jnp.transpose(
        out.reshape(B // E, C2, D, E, S), (0, 3, 2, 4, 1)).reshape(
            B, D, H, W, C2)
    return out


def kernel(x, w_b1_0, b_b1_0, w_b1_1, b_b1_1, w_b1_2, b_b1_2,
           w_b2_0, b_b2_0, w_b2_1, b_b2_1, w_b2_2, b_b2_2):
    layers = [(w_b1_0, b_b1_0), (w_b1_1, b_b1_1), (w_b1_2, b_b1_2),
              (w_b2_0, b_b2_0), (w_b2_1, b_b2_1), (w_b2_2, b_b2_2)]
    return _forward(x, layers)


# speedup vs baseline: 3.7745x; 1.0017x over previous
import functools

import numpy as np
import jax
import jax.numpy as jnp
from jax.experimental import pallas as pl
from jax.experimental.pallas import tpu as pltpu


def _round_up(n, m):
    return ((n + m - 1) // m) * m


def _compact_kernel(x_ref, w_ref, b_ref, mw_ref, mo_ref, o_ref, z_ref, *,
                    E, N, S, W, D, CH, C2, KD, KH, KW, layers_per_block,
                    n_blocks):
    SL = E * S
    NT = D * SL
    pd, ph, pw = KD // 2, KH // 2, KW // 2

    def roll(v, s, width):
        if s % width == 0:
            return v
        return pltpu.roll(v, (-s) % width, axis=1)

    mw = [mw_ref[k:k + 1, :] for k in range(KW)]
    mo = [mo_ref[k:k + 1, :] for k in range(KH)]

    for kd in range(KD):
        if kd < pd:
            z_ref[kd * KW * CH:(kd + 1) * KW * CH, :(pd - kd) * SL] = (
                jnp.zeros((KW * CH, (pd - kd) * SL), jnp.bfloat16))
        elif kd > pd:
            z_ref[kd * KW * CH:(kd + 1) * KW * CH, NT - (kd - pd) * SL:] = (
                jnp.zeros((KW * CH, (kd - pd) * SL), jnp.bfloat16))

    h = jnp.concatenate(
        [x_ref[...].astype(jnp.float32),
         jnp.zeros((CH - C2, NT), jnp.float32)], axis=0)

    li = 0
    for _blk in range(n_blocks):
        h_in = h
        for _l in range(layers_per_block):
            hb = h.astype(jnp.bfloat16)
            for src in range(D):
                sl = hb[:, src * SL:(src + 1) * SL]
                for kw in range(KW):
                    if kw == pw:
                        u = sl
                    else:
                        u = roll(sl, kw - pw, SL) * mw[kw]
                    row = kw * CH
                    for kd in range(KD):
                        d = src - kd + pd
                        if 0 <= d < D:
                            z_ref[kd * KW * CH + row:
                                  kd * KW * CH + row + CH,
                                  d * SL:(d + 1) * SL] = u
            r = jnp.dot(w_ref[li], z_ref[...],
                        preferred_element_type=jnp.float32)
            acc = None
            for kh in range(KH):
                part = roll(r[kh * CH:(kh + 1) * CH, :], (kh - ph) * W, NT)
                if kh != ph:
                    part = part * mo[kh]
                acc = part if acc is None else acc + part
            h = jnp.maximum(acc + b_ref[li], 0.0)
            li += 1
        h = h + h_in
    o_ref[...] = h[:C2, :]


def _forward(x, layers):
    B, D, H, W, C2 = x.shape
    KD, KH, KW = layers[0][0].shape[:3]
    pd, ph, pw = KD // 2, KH // 2, KW // 2
    CH = _round_up(max(max(w.shape[3], w.shape[4]) for w, _ in layers), 8)
    S, N = H * W, D * H * W
    E = next((e for e in (8, 16, 4, 2, 1)
              if B % e == 0 and (e * S) % 128 == 0), 1)
    SL = E * S
    NT = D * SL

    x_flat = jnp.transpose(
        x.astype(jnp.bfloat16).reshape(B // E, E, D, S, C2),
        (0, 4, 2, 1, 3)).reshape(B // E, C2, NT)

    w_stack, b_stack = [], []
    for (w, b) in layers:
        cin, cout = w.shape[3], w.shape[4]
        wpd = jnp.zeros((KD, KH, KW, CH, CH), jnp.float32)
        wpd = wpd.at[..., :cin, :cout].set(w)
        w_stack.append(
            jnp.transpose(wpd, (1, 4, 0, 2, 3)).reshape(KH * CH, KD * KW * CH))
        b_stack.append(
            jnp.zeros((CH,), jnp.float32).at[:cout].set(b).reshape(CH, 1))
    w_stack = jnp.stack(w_stack).astype(jnp.bfloat16)
    b_stack = jnp.stack(b_stack)

    qs = np.arange(SL)
    qw_s = qs % W
    mw = np.zeros((8, SL), np.float32)
    for kw in range(KW):
        mw[kw] = (qw_s + kw - pw >= 0) & (qw_s + kw - pw < W)
    q = np.arange(NT)
    qh = (q % S) // W
    mo = np.zeros((8, NT), np.float32)
    for kh in range(KH):
        mo[kh] = (qh + kh - ph >= 0) & (qh + kh - ph < H)
    mw = jnp.asarray(mw, jnp.bfloat16)
    mo = jnp.asarray(mo)

    kfn = functools.partial(
        _compact_kernel, E=E, N=N, S=S, W=W, D=D, CH=CH, C2=C2, KD=KD,
        KH=KH, KW=KW, layers_per_block=len(layers) // 2, n_blocks=2)

    out = pl.pallas_call(
        kfn,
        out_shape=jax.ShapeDtypeStruct((B // E, C2, NT), jnp.float32),
        grid_spec=pltpu.PrefetchScalarGridSpec(
            num_scalar_prefetch=0,
            grid=(B // E,),
            in_specs=[
                pl.BlockSpec((None, C2, NT), lambda g: (g, 0, 0)),
                pl.BlockSpec(w_stack.shape, lambda g: (0, 0, 0)),
                pl.BlockSpec(b_stack.shape, lambda g: (0, 0, 0)),
                pl.BlockSpec(mw.shape, lambda g: (0, 0)),
                pl.BlockSpec(mo.shape, lambda g: (0, 0)),
            ],
            out_specs=pl.BlockSpec((None, C2, NT), lambda g: (g, 0, 0)),
            scratch_shapes=[
                pltpu.VMEM((KD * KW * CH, NT), jnp.bfloat16)],
        ),
        compiler_params=pltpu.CompilerParams(
            dimension_semantics=("parallel",),
            vmem_limit_bytes=56 * 1024 * 1024),
    )(x_flat, w_stack, b_stack, mw, mo)

    out = jnp.transpose(
        out.reshape(B // E, C2, D, E, S), (0, 3, 2, 4, 1)).reshape(
            B, D, H, W, C2)
    return out


def kernel(x, w_b1_0, b_b1_0, w_b1_1, b_b1_1, w_b1_2, b_b1_2,
           w_b2_0, b_b2_0, w_b2_1, b_b2_1, w_b2_2, b_b2_2):
    layers = [(w_b1_0, b_b1_0), (w_b1_1, b_b1_1), (w_b1_2, b_b1_2),
              (w_b2_0, b_b2_0), (w_b2_1, b_b2_1), (w_b2_2, b_b2_2)]
    return _forward(x, layers)
